# baseline (device time: 33335 ns/iter reference)
import jax
import jax.numpy as jnp
from jax import lax
from jax.experimental import pallas as pl
from jax.experimental.pallas import tpu as pltpu

N_DEV = 8
B, D, H = 512, 256, 512
R = B // N_DEV
N_LAYERS = 3
MESH = pl.DeviceIdType.MESH
WIRE = jnp.bfloat16

GROUPS = ((0, 1), (1, 3), (3, 5), (5, 7), (7, 8))


def kernel(x, Win0, Wout0, Win1, Wout1, Win2, Wout2):
    def body(x_hbm, win0_hbm, wout0_hbm, win1_hbm, wout1_hbm, win2_hbm,
             wout2_hbm, out_ref,
             x_vmem, win_vmem, wout_vmem,
             partial_ref, rs_buf, x_buf, load_sems, send_sems, recv_sems):
        my = lax.axis_index("i")

        def dev(idx):
            return (lax.rem(idx + N_DEV, N_DEV),)

        load_x = pltpu.make_async_copy(x_hbm, x_vmem, load_sems.at[0])
        load_x.start()
        win_hbms = [win0_hbm, win1_hbm, win2_hbm]
        wout_hbms = [wout0_hbm, wout1_hbm, wout2_hbm]
        w_loads = []
        for l in range(N_LAYERS):
            lw = pltpu.make_async_copy(win_hbms[l], win_vmem.at[l],
                                       load_sems.at[1 + 2 * l])
            lo = pltpu.make_async_copy(wout_hbms[l], wout_vmem.at[l],
                                       load_sems.at[2 + 2 * l])
            lw.start()
            lo.start()
            w_loads.append((lw, lo))

        barrier_sem = pltpu.get_barrier_semaphore()
        for k in range(1, N_DEV):
            pl.semaphore_signal(barrier_sem, inc=1, device_id=dev(my + k),
                                device_id_type=MESH)
        pl.semaphore_wait(barrier_sem, N_DEV - 1)

        w_cast = {}

        def mlp(xv, l):
            if l not in w_cast:
                lw, lo = w_loads[l]
                lw.wait()
                lo.wait()
                w_cast[l] = (win_vmem[l].astype(WIRE),
                             wout_vmem[l].astype(WIRE))
            wb, ob = w_cast[l]
            hv = jnp.dot(xv.astype(WIRE), wb,
                         preferred_element_type=jnp.float32)
            hv = jnp.maximum(hv, 0.0)
            return jnp.dot(hv.astype(WIRE), ob,
                           preferred_element_type=jnp.float32)

        def rs_recv_wait(l, slot):
            pltpu.make_async_remote_copy(
                src_ref=partial_ref.at[l, pl.ds(0, 1)],
                dst_ref=rs_buf.at[l, pl.ds(slot, 1)],
                send_sem=send_sems.at[l, 0, slot],
                recv_sem=recv_sems.at[l, 0, slot],
                device_id=dev(my), device_id_type=MESH,
            ).wait_recv()

        def finish_layer(l, own_f32):
            for j in range(1, N_DEV):
                rs_recv_wait(l, j - 1)
            acc = own_f32
            for s in range(N_DEV - 1):
                acc = acc + rs_buf[l, s].astype(jnp.float32)
            return acc

        load_x.wait()
        p0 = mlp(x_vmem[:, :], 0)
        partial_ref[0] = p0.astype(WIRE).reshape(N_DEV, R, D)
        for k in range(1, N_DEV):
            tgt = lax.rem(my + k, N_DEV)
            pltpu.make_async_remote_copy(
                src_ref=partial_ref.at[0, pl.ds(tgt, 1)],
                dst_ref=rs_buf.at[0, pl.ds(k - 1, 1)],
                send_sem=send_sems.at[0, 0, k - 1],
                recv_sem=recv_sems.at[0, 0, k - 1],
                device_id=dev(my + k), device_id_type=MESH,
            ).start()
        own0 = partial_ref[0, pl.ds(my, 1)][0].astype(jnp.float32)
        acc = finish_layer(0, own0)

        for l in range(1, N_LAYERS):
            x_buf[l - 1, 0] = acc.astype(WIRE)
            for k in range(1, N_DEV):
                pltpu.make_async_remote_copy(
                    src_ref=x_buf.at[l - 1, pl.ds(0, 1)],
                    dst_ref=x_buf.at[l - 1, pl.ds(k, 1)],
                    send_sem=send_sems.at[l - 1, 1, k - 1],
                    recv_sem=recv_sems.at[l - 1, 1, k - 1],
                    device_id=dev(my + k), device_id_type=MESH,
                ).start()

            own_f32 = None
            for (s, e) in GROUPS:
                for j in range(max(s, 1), e):
                    pltpu.make_async_remote_copy(
                        src_ref=x_buf.at[l - 1, pl.ds(0, 1)],
                        dst_ref=x_buf.at[l - 1, pl.ds(j, 1)],
                        send_sem=send_sems.at[l - 1, 1, j - 1],
                        recv_sem=recv_sems.at[l - 1, 1, j - 1],
                        device_id=dev(my - j), device_id_type=MESH,
                    ).wait_recv()
                xg = x_buf[l - 1, s:e].reshape((e - s) * R, D)
                pg = mlp(xg, l)
                partial_ref[l, s:e] = pg.astype(WIRE).reshape(e - s, R, D)
                if s == 0:
                    own_f32 = pg[0:R]
                for j in range(max(s, 1), e):
                    pltpu.make_async_remote_copy(
                        src_ref=partial_ref.at[l, pl.ds(j, 1)],
                        dst_ref=rs_buf.at[l, pl.ds(j - 1, 1)],
                        send_sem=send_sems.at[l, 0, j - 1],
                        recv_sem=recv_sems.at[l, 0, j - 1],
                        device_id=dev(my - j), device_id_type=MESH,
                    ).start()

            acc = finish_layer(l, own_f32)

        out_ref[:, :] = acc

        for l in range(N_LAYERS):
            for k in range(1, N_DEV):
                pltpu.make_async_remote_copy(
                    src_ref=partial_ref.at[l, pl.ds(0, 1)],
                    dst_ref=rs_buf.at[l, pl.ds(0, 1)],
                    send_sem=send_sems.at[l, 0, k - 1],
                    recv_sem=recv_sems.at[l, 0, k - 1],
                    device_id=dev(my), device_id_type=MESH,
                ).wait_send()
        for l in range(N_LAYERS - 1):
            for k in range(1, N_DEV):
                pltpu.make_async_remote_copy(
                    src_ref=x_buf.at[l, pl.ds(0, 1)],
                    dst_ref=x_buf.at[l, pl.ds(0, 1)],
                    send_sem=send_sems.at[l, 1, k - 1],
                    recv_sem=recv_sems.at[l, 1, k - 1],
                    device_id=dev(my), device_id_type=MESH,
                ).wait_send()

    return pl.pallas_call(
        body,
        out_shape=jax.ShapeDtypeStruct((R, D), jnp.float32),
        in_specs=[pl.BlockSpec(memory_space=pl.ANY)] * 7,
        out_specs=pl.BlockSpec(memory_space=pltpu.VMEM),
        scratch_shapes=[
            pltpu.VMEM((B, D), jnp.float32),
            pltpu.VMEM((N_LAYERS, D, H), jnp.float32),
            pltpu.VMEM((N_LAYERS, H, D), jnp.float32),
            pltpu.VMEM((N_LAYERS, N_DEV, R, D), WIRE),
            pltpu.VMEM((N_LAYERS, N_DEV - 1, R, D), WIRE),
            pltpu.VMEM((N_LAYERS - 1, N_DEV, R, D), WIRE),
            pltpu.SemaphoreType.DMA((1 + 2 * N_LAYERS,)),
            pltpu.SemaphoreType.DMA((N_LAYERS, 2, N_DEV - 1)),
            pltpu.SemaphoreType.DMA((N_LAYERS, 2, N_DEV - 1)),
        ],
        compiler_params=pltpu.CompilerParams(collective_id=0),
    )(x, Win0, Wout0, Win1, Wout1, Win2, Wout2)
